# baseline (speedup 1.0000x reference)
import functools

import jax
import jax.numpy as jnp
from jax.experimental import pallas as pl
from jax.experimental.pallas import tpu as pltpu

_ENC_K = (32, 16, 8, 4)
_DEC_K = (4, 8, 16, 32)


def _conv_body(xs_ref, w_ref, sb_ref, o_ref, xr_ref, *, taps, cdepth, co4,
               oh, ws, nb, relu):
    ll = oh * ws
    lt = ll + taps - 1
    bb = xs_ref.shape[0]
    for b in range(bb):
        for t in range(taps):
            xr_ref[t * cdepth:(t + 1) * cdepth, :] = (
                xs_ref[b, :, t * ws: t * ws + lt])
        n0 = 0
        while n0 < ll:
            nw = min(nb, ll - n0)
            pc = jnp.dot(w_ref[...], xr_ref[:, n0: n0 + nw + taps - 1],
                         preferred_element_type=jnp.float32)
            acc = pc[0:co4, 0:nw]
            for c in range(1, taps):
                acc = acc + pc[c * co4:(c + 1) * co4, c: c + nw]
            y = acc * sb_ref[:, 0:1] + sb_ref[:, 1:2]
            if relu:
                y = jnp.maximum(y, 0.0)
            o_ref[b, :, n0: n0 + nw] = y.astype(o_ref.dtype)
            n0 += nw


def _conv_call(xs, wm, sb, *, taps, oh, ws, relu, out_dtype, bb):
    B, C, FL = xs.shape
    M, K = wm.shape
    co4 = M // taps
    ll = oh * ws
    lt = ll + taps - 1
    nb = max(128, (600_000 // (M * 4)) // 128 * 128)
    body = functools.partial(_conv_body, taps=taps, cdepth=C, co4=co4,
                             oh=oh, ws=ws, nb=nb, relu=relu)
    return pl.pallas_call(
        body,
        out_shape=jax.ShapeDtypeStruct((B, co4, ll), out_dtype),
        grid=(B // bb,),
        in_specs=[
            pl.BlockSpec((bb, C, FL), lambda i: (i, 0, 0)),
            pl.BlockSpec((M, K), lambda i: (0, 0)),
            pl.BlockSpec((co4, 2), lambda i: (0, 0)),
        ],
        out_specs=pl.BlockSpec((bb, co4, ll), lambda i: (i, 0, 0)),
        scratch_shapes=[pltpu.VMEM((K, lt), jnp.bfloat16)],
        compiler_params=pltpu.CompilerParams(
            dimension_semantics=("parallel",)),
    )(xs, wm, sb)


def _mlp_body(z_ref, w1_ref, sb1_ref, w2_ref, b2_ref, o_ref):
    h = jnp.dot(z_ref[...], w1_ref[...], preferred_element_type=jnp.float32)
    h = h * sb1_ref[0:1, :] + sb1_ref[1:2, :]
    h = jnp.maximum(h, 0.0).astype(w2_ref.dtype)
    o_ref[...] = jnp.dot(h, w2_ref[...],
                         preferred_element_type=jnp.float32) + b2_ref[...]


def _mlp_call(z, w1t, sb1, w2t, b2):
    B = z.shape[0]
    gb = B // 2 if B % 2 == 0 else B
    return pl.pallas_call(
        _mlp_body,
        out_shape=jax.ShapeDtypeStruct((B, w2t.shape[1]), jnp.float32),
        grid=(B // gb,),
        in_specs=[
            pl.BlockSpec((gb, w1t.shape[0]), lambda i: (i, 0)),
            pl.BlockSpec(w1t.shape, lambda i: (0, 0)),
            pl.BlockSpec(sb1.shape, lambda i: (0, 0)),
            pl.BlockSpec(w2t.shape, lambda i: (0, 0)),
            pl.BlockSpec(b2.shape, lambda i: (0, 0)),
        ],
        out_specs=pl.BlockSpec((gb, w2t.shape[1]), lambda i: (i, 0)),
        compiler_params=pltpu.CompilerParams(
            dimension_semantics=("parallel",)),
    )(z, w1t, sb1, w2t, b2)


def _wm(w_rows):
    T, co4, K = w_rows.shape
    C = K // T
    return w_rows.reshape(T, co4, T, C).transpose(2, 1, 0, 3).reshape(
        T * co4, T * C)


def _enc_fold(x, kh):
    p = kh - 1
    xp = jnp.pad(x, ((0, 0), (0, 0), (p, p), (p, p)))
    B, C, H2, W2 = xp.shape
    hs, ws = H2 // 2, W2 // 2
    xs = xp.reshape(B, C, hs, 2, ws, 2).transpose(0, 3, 5, 1, 2, 4)
    xs = xs.reshape(B, 4 * C, hs * ws)
    return jnp.pad(xs, ((0, 0), (0, 0), (0, kh)))


def _dec_fold(x, khp):
    q = (khp - 1) // 2
    xp = jnp.pad(x, ((0, 0), (0, 0), (q, q), (q, q)))
    B, C, hs, ws = xp.shape
    return jnp.pad(xp.reshape(B, C, hs * ws), ((0, 0), (0, 0), (0, khp)))


def _crop(yflat, oh, ws, ow):
    B, C, _ = yflat.shape
    return yflat.reshape(B, C, oh, ws)[:, :, :, :ow]


def _shuffle(y4, H, W):
    B, c4, _, _ = y4.shape
    co = c4 // 4
    y = y4.reshape(B, 2, 2, co, H, W).transpose(0, 3, 4, 1, 5, 2)
    return y.reshape(B, co, 2 * H, 2 * W)


def kernel(x,
           enc0_w_rows, enc0_sb, enc1_w_rows, enc1_sb,
           enc2_w_rows, enc2_sb, enc3_w_rows, enc3_sb,
           dec0_w_rows, dec0_sb, dec1_w_rows, dec1_sb,
           dec2_w_rows, dec2_sb, dec3_w_rows, dec3_sb,
           mlp_w1t, mlp_sb1, mlp_w2t, mlp_b2):
    B = x.shape[0]
    bb = next(b for b in (4, 2, 1) if B % b == 0)
    h = x.astype(jnp.bfloat16)

    enc = [(enc0_w_rows, enc0_sb), (enc1_w_rows, enc1_sb),
           (enc2_w_rows, enc2_sb), (enc3_w_rows, enc3_sb)]
    for (w_r, sb), k in zip(enc, _ENC_K):
        kh = k // 2
        oh, ow = h.shape[2] // 2, h.shape[3] // 2
        ws = ow + kh - 1
        xs = _enc_fold(h, kh)
        y = _conv_call(xs, _wm(w_r), sb, taps=kh, oh=oh, ws=ws, relu=True,
                       out_dtype=jnp.bfloat16, bb=bb)
        h = _crop(y, oh, ws, ow)

    z = h.reshape(B, -1)
    zo = _mlp_call(z, mlp_w1t, mlp_sb1, mlp_w2t, mlp_b2)
    h = zo.reshape(B, 64, 8, 8).astype(jnp.bfloat16)

    dec = [(dec0_w_rows, dec0_sb), (dec1_w_rows, dec1_sb),
           (dec2_w_rows, dec2_sb), (dec3_w_rows, dec3_sb)]
    for i, ((w_r, sb), k) in enumerate(zip(dec, _DEC_K)):
        khp = k // 2 + 1
        kh = k // 2
        H, W = h.shape[2], h.shape[3]
        ws = W + kh
        relu = i < 3
        xs = _dec_fold(h, khp)
        y = _conv_call(xs, _wm(w_r), sb, taps=khp, oh=H, ws=ws, relu=relu,
                       out_dtype=jnp.bfloat16 if relu else jnp.float32, bb=bb)
        h = _shuffle(_crop(y, H, ws, W), H, W)
    return h


# speedup vs baseline: 1.8546x; 1.8546x over previous
import functools

import jax
import jax.numpy as jnp
from jax.experimental import pallas as pl
from jax.experimental.pallas import tpu as pltpu

_ENC_K = (32, 16, 8, 4)
_DEC_K = (4, 8, 16, 32)


def _conv_body(xf_ref, w_ref, sb_ref, o_ref, *, taps, co4, row_stride,
               out_len, relu):
    bb = xf_ref.shape[0]
    for b in range(bb):
        pc = jnp.dot(w_ref[...], xf_ref[b],
                     preferred_element_type=jnp.float32)
        acc = pc[0:co4, 0:out_len]
        for a in range(1, taps):
            acc = acc + pc[a * co4:(a + 1) * co4,
                           a * row_stride: a * row_stride + out_len]
        y = acc * sb_ref[:, 0:1] + sb_ref[:, 1:2]
        if relu:
            y = jnp.maximum(y, 0.0)
        o_ref[b] = y.astype(o_ref.dtype)


def _conv_call(xf, w_rows, sb, *, row_stride, out_len, relu, out_dtype, bb):
    B, K, L = xf.shape
    taps, co4, k2 = w_rows.shape
    wm = w_rows.reshape(taps * co4, K)
    body = functools.partial(_conv_body, taps=taps, co4=co4,
                             row_stride=row_stride, out_len=out_len,
                             relu=relu)
    return pl.pallas_call(
        body,
        out_shape=jax.ShapeDtypeStruct((B, co4, out_len), out_dtype),
        grid=(B // bb,),
        in_specs=[
            pl.BlockSpec((bb, K, L), lambda i: (i, 0, 0)),
            pl.BlockSpec((taps * co4, K), lambda i: (0, 0)),
            pl.BlockSpec((co4, 2), lambda i: (0, 0)),
        ],
        out_specs=pl.BlockSpec((bb, co4, out_len), lambda i: (i, 0, 0)),
        compiler_params=pltpu.CompilerParams(
            dimension_semantics=("parallel",)),
    )(xf, wm, sb)


def _mlp_body(z_ref, w1_ref, sb1_ref, w2_ref, b2_ref, o_ref):
    h = jnp.dot(z_ref[...], w1_ref[...], preferred_element_type=jnp.float32)
    h = h * sb1_ref[0:1, :] + sb1_ref[1:2, :]
    h = jnp.maximum(h, 0.0).astype(w2_ref.dtype)
    o_ref[...] = jnp.dot(h, w2_ref[...],
                         preferred_element_type=jnp.float32) + b2_ref[...]


def _mlp_call(z, w1t, sb1, w2t, b2):
    B = z.shape[0]
    gb = B // 2 if B % 2 == 0 else B
    return pl.pallas_call(
        _mlp_body,
        out_shape=jax.ShapeDtypeStruct((B, w2t.shape[1]), jnp.float32),
        grid=(B // gb,),
        in_specs=[
            pl.BlockSpec((gb, w1t.shape[0]), lambda i: (i, 0)),
            pl.BlockSpec(w1t.shape, lambda i: (0, 0)),
            pl.BlockSpec(sb1.shape, lambda i: (0, 0)),
            pl.BlockSpec(w2t.shape, lambda i: (0, 0)),
            pl.BlockSpec(b2.shape, lambda i: (0, 0)),
        ],
        out_specs=pl.BlockSpec((gb, w2t.shape[1]), lambda i: (i, 0)),
        compiler_params=pltpu.CompilerParams(
            dimension_semantics=("parallel",)),
    )(z, w1t, sb1, w2t, b2)


def _unshuffle(x):
    B, C, H2, W2 = x.shape
    H, W = H2 // 2, W2 // 2
    x = x.reshape(B, C, H, 2, W, 2).transpose(0, 3, 5, 1, 2, 4)
    return x.reshape(B, 4 * C, H, W)


def _fold_w(x, taps, out_w):
    B, C, H, W = x.shape
    q = jnp.concatenate([x[:, :, :, t:t + out_w] for t in range(taps)], axis=1)
    return q.reshape(B, taps * C, H * out_w)


def kernel(x,
           enc0_w_rows, enc0_sb, enc1_w_rows, enc1_sb,
           enc2_w_rows, enc2_sb, enc3_w_rows, enc3_sb,
           dec0_w_rows, dec0_sb, dec1_w_rows, dec1_sb,
           dec2_w_rows, dec2_sb, dec3_w_rows, dec3_sb,
           mlp_w1t, mlp_sb1, mlp_w2t, mlp_b2):
    B = x.shape[0]
    bb = next(b for b in (4, 2, 1) if B % b == 0)
    h = x.astype(jnp.bfloat16)

    enc = [(enc0_w_rows, enc0_sb), (enc1_w_rows, enc1_sb),
           (enc2_w_rows, enc2_sb), (enc3_w_rows, enc3_sb)]
    for (w_r, sb), k in zip(enc, _ENC_K):
        kh = k // 2
        pad = kh - 1
        oh, ow = h.shape[2] // 2, h.shape[3] // 2
        xp = jnp.pad(h, ((0, 0), (0, 0), (pad, pad), (pad, pad)))
        xf = _fold_w(_unshuffle(xp), kh, ow).astype(jnp.bfloat16)
        y = _conv_call(xf, w_r, sb, row_stride=ow, out_len=oh * ow,
                       relu=True, out_dtype=jnp.bfloat16, bb=bb)
        h = y.reshape(B, -1, oh, ow)

    z = h.reshape(B, -1)
    zo = _mlp_call(z, mlp_w1t, mlp_sb1, mlp_w2t, mlp_b2)
    h = zo.reshape(B, 64, 8, 8).astype(jnp.bfloat16)

    dec = [(dec0_w_rows, dec0_sb), (dec1_w_rows, dec1_sb),
           (dec2_w_rows, dec2_sb), (dec3_w_rows, dec3_sb)]
    for i, ((w_r, sb), k) in enumerate(zip(dec, _DEC_K)):
        khp = k // 2 + 1
        q = (khp - 1) // 2
        H, W = h.shape[2], h.shape[3]
        relu = i < 3
        xp = jnp.pad(h, ((0, 0), (0, 0), (q, q), (q, q)))
        xf = _fold_w(xp, khp, W).astype(jnp.bfloat16)
        y = _conv_call(xf, w_r, sb, row_stride=W, out_len=H * W, relu=relu,
                       out_dtype=jnp.bfloat16 if relu else jnp.float32, bb=bb)
        co = y.shape[1] // 4
        y = y.reshape(B, 2, 2, co, H, W).transpose(0, 3, 4, 1, 5, 2)
        h = y.reshape(B, co, 2 * H, 2 * W)
    return h
